# Initial kernel scaffold; baseline (speedup 1.0000x reference)
import jax
import jax.numpy as jnp
from jax.experimental import pallas as pl


def kernel(x_user, x_item, edge_rates, edge_rated_by, W_in_user, b_in_user, W_in_item, b_in_item, Wself0_ui, Wneigh0_ui, b0_ui, Wself0_iu, Wneigh0_iu, b0_iu, Wself1_ui, Wneigh1_ui, b1_ui, Wself1_iu, Wneigh1_iu, b1_iu, ln0_g, ln0_b):
    raise NotImplementedError("write your pallas kernel here")


# speedup vs baseline: 1.7833x; 1.7833x over previous
import functools

import jax
import jax.numpy as jnp
from jax import lax
from jax.experimental import pallas as pl
from jax.experimental.pallas import tpu as pltpu
from jax.experimental.pallas import tpu_sc as plsc

N = 10000
D = 128
E = 320000

DEGW = 16
NC = 2
NS = 16
CH = 128
EP = 321536
TPC = EP // (NS * CH)
NP = 3
PB = 3336
PBP = PB + 8
JUNK = PB
NPB = NP * PB
SPT = 216
ZR = SPT

_mesh = plsc.VectorSubcoreMesh(core_axis_name="c", subcore_axis_name="s",
                               num_cores=NC)


def _make_segsum(with_deg):
  out_types = [jax.ShapeDtypeStruct((2 * NPB, D), jnp.float32)]
  if with_deg:
    out_types.append(jax.ShapeDtypeStruct((2 * NPB, D), jnp.float32))

  scratch = [
      pltpu.VMEM((TPC, CH), jnp.int32),
      pltpu.VMEM((TPC, CH), jnp.int32),
      pltpu.VMEM((CH,), jnp.int32),
      pltpu.VMEM((CH,), jnp.int32),
      pltpu.VMEM((CH, D), jnp.float32),
      pltpu.VMEM_SHARED((PBP, D), jnp.float32),
  ]
  if with_deg:
    scratch += [
        pltpu.VMEM((CH, D), jnp.float32),
    ]
  scratch.append(pltpu.SemaphoreType.DMA)

  def body(hu_hbm, hi_hbm, sui, dui, siu, diu, z128, z16, *rest):
    if with_deg:
      out_sum, out_deg = rest[0], rest[1]
      rest = rest[2:]
      src_v, dst_v, i1, i2, rows, accum, ones, sem = rest
    else:
      out_sum = rest[0]
      out_deg = None
      src_v, dst_v, i1, i2, rows, accum, sem = rest[1:]
      ones = None

    c = lax.axis_index("c")
    s = lax.axis_index("s")
    one16 = jnp.full((16,), 1.0, jnp.float32)

    if with_deg:
      @pl.loop(0, CH)
      def _(r):
        @pl.loop(0, D, step=16)
        def _(cc):
          ones[r, pl.ds(cc, 16)] = one16

    stripe = jnp.minimum(s * SPT, PB - SPT)


    def run(src3d, dst4d, h_hbm):
      pltpu.sync_copy(src3d.at[s], src_v)

      @pl.loop(0, NP)
      def _(p):
        pltpu.sync_copy(z128, accum.at[pl.ds(stripe, ZR)])
        pltpu.sync_copy(dst4d.at[p * NS + s], dst_v)
        plsc.subcore_barrier()

        @pl.loop(0, TPC)
        def _(j):
          for k in range(0, CH, 16):
            i1[pl.ds(k, 16)] = src_v[j, pl.ds(k, 16)]
            i2[pl.ds(k, 16)] = dst_v[j, pl.ds(k, 16)]
          pltpu.async_copy(h_hbm.at[i1], rows, sem).wait()
          pltpu.sync_copy(rows, accum.at[i2], add=True)

        plsc.subcore_barrier()
        pltpu.sync_copy(accum.at[pl.ds(stripe, SPT)],
                        out_sum.at[pl.ds(c * NPB + p * PB + stripe, SPT)])
        plsc.subcore_barrier()

      if with_deg:
        @pl.loop(0, NP)
        def _(p):
          pltpu.sync_copy(z128, accum.at[pl.ds(stripe, ZR)])
          pltpu.sync_copy(dst4d.at[p * NS + s], dst_v)
          plsc.subcore_barrier()

          @pl.loop(0, TPC)
          def _(j):
            for k in range(0, CH, 16):
              i2[pl.ds(k, 16)] = dst_v[j, pl.ds(k, 16)]
            pltpu.sync_copy(ones, accum.at[i2], add=True)

          plsc.subcore_barrier()
          pltpu.sync_copy(accum.at[pl.ds(stripe, SPT)],
                          out_deg.at[pl.ds(c * NPB + p * PB + stripe, SPT)])
          plsc.subcore_barrier()

    @pl.when(c == 0)
    def _():
      run(sui, dui, hu_hbm)

    @pl.when(c == 1)
    def _():
      run(siu, diu, hi_hbm)

  return pl.kernel(body, out_type=out_types, mesh=_mesh,
                   scratch_types=scratch)


_segsum_deg = _make_segsum(True)
_segsum = _make_segsum(False)


RB = 1000


def _proj_body(x_ref, w_ref, b_ref, o_ref):
  o_ref[...] = (jnp.dot(x_ref[...], w_ref[...],
                        preferred_element_type=jnp.float32) + b_ref[...])


def _proj(x, w, b):
  return pl.pallas_call(
      _proj_body,
      grid=(N // RB,),
      in_specs=[
          pl.BlockSpec((RB, D), lambda i: (i, 0)),
          pl.BlockSpec((D, D), lambda i: (0, 0)),
          pl.BlockSpec((D,), lambda i: (0,)),
      ],
      out_specs=pl.BlockSpec((RB, D), lambda i: (i, 0)),
      out_shape=jax.ShapeDtypeStruct((N, D), jnp.float32),
  )(x, w, b)


def _make_combine(apply_ln):
  def body(h_ref, summ_ref, deg_ref, ws_ref, wn_ref, b_ref, g_ref, bb_ref,
           o_ref):
    inv = 1.0 / jnp.maximum(deg_ref[...][:, 0:1], 1.0)
    h = (jnp.dot(h_ref[...], ws_ref[...], preferred_element_type=jnp.float32)
         + jnp.dot(summ_ref[...] * inv, wn_ref[...],
                   preferred_element_type=jnp.float32)
         + b_ref[...])
    if apply_ln:
      mu = jnp.mean(h, axis=-1, keepdims=True)
      var = jnp.mean((h - mu) ** 2, axis=-1, keepdims=True)
      h = g_ref[...] * (h - mu) / jnp.sqrt(var + 1e-5) + bb_ref[...]
      h = jnp.maximum(h, 0.0)
    o_ref[...] = h

  def call(h, summ, deg, ws, wn, b, g, bb):
    return pl.pallas_call(
        body,
        grid=(N // RB,),
        in_specs=[
            pl.BlockSpec((RB, D), lambda i: (i, 0)),
            pl.BlockSpec((RB, D), lambda i: (i, 0)),
            pl.BlockSpec((RB, D), lambda i: (i, 0)),
            pl.BlockSpec((D, D), lambda i: (0, 0)),
            pl.BlockSpec((D, D), lambda i: (0, 0)),
            pl.BlockSpec((D,), lambda i: (0,)),
            pl.BlockSpec((D,), lambda i: (0,)),
            pl.BlockSpec((D,), lambda i: (0,)),
        ],
        out_specs=pl.BlockSpec((RB, D), lambda i: (i, 0)),
        out_shape=jax.ShapeDtypeStruct((N, D), jnp.float32),
    )(h, summ, deg, ws, wn, b, g, bb)

  return call


_combine_ln = _make_combine(True)
_combine = _make_combine(False)


@jax.jit
def kernel(x_user, x_item, edge_rates, edge_rated_by,
           W_in_user, b_in_user, W_in_item, b_in_item,
           Wself0_ui, Wneigh0_ui, b0_ui, Wself0_iu, Wneigh0_iu, b0_iu,
           Wself1_ui, Wneigh1_ui, b1_ui, Wself1_iu, Wneigh1_iu, b1_iu,
           ln0_g, ln0_b):
  def pad(d, fill):
    return jnp.concatenate([d, jnp.full((EP - E,), fill, jnp.int32)])

  def remap(d):
    d = pad(d, N)
    slabs = [jnp.where((d >= k * PB) & (d < (k + 1) * PB), d - k * PB, JUNK)
             for k in range(NP)]
    return jnp.stack(slabs).reshape(NP * NS, TPC, CH)

  sui = pad(edge_rates[0], 0).reshape(NS, TPC, CH)
  dui = remap(edge_rates[1])
  siu = pad(edge_rated_by[0], 0).reshape(NS, TPC, CH)
  diu = remap(edge_rated_by[1])

  hu = _proj(x_user, W_in_user, b_in_user)
  hi = _proj(x_item, W_in_item, b_in_item)

  z128 = jnp.zeros((ZR, D), jnp.float32)
  z16 = jnp.zeros((ZR, DEGW), jnp.float32)

  summ0, deg = _segsum_deg(hu, hi, sui, dui, siu, diu, z128, z16)
  deg_i, deg_u = deg[:N], deg[NPB:NPB + N]

  hi_r = _combine_ln(hi, summ0[:N], deg_i, Wself0_ui, Wneigh0_ui, b0_ui,
                     ln0_g, ln0_b)
  hu_r = _combine_ln(hu, summ0[NPB:NPB + N], deg_u, Wself0_iu, Wneigh0_iu,
                     b0_iu, ln0_g, ln0_b)

  summ1, = _segsum(hu_r, hi_r, sui, dui, siu, diu, z128, z16)

  hi2 = _combine(hi_r, summ1[:N], deg_i, Wself1_ui, Wneigh1_ui, b1_ui,
                 ln0_g, ln0_b)
  hu2 = _combine(hu_r, summ1[NPB:NPB + N], deg_u, Wself1_iu, Wneigh1_iu,
                 b1_iu, ln0_g, ln0_b)
  return (hu2, hi2)


# speedup vs baseline: 2.7273x; 1.5293x over previous
import functools

import jax
import jax.numpy as jnp
from jax import lax
from jax.experimental import pallas as pl
from jax.experimental.pallas import tpu as pltpu
from jax.experimental.pallas import tpu_sc as plsc

N = 10000
D = 128
E = 320000

DEGW = 16
NC = 2
NS = 16
CH = 128
EP = 321536
TPC = EP // (NS * CH)
NP = 2
PB = 5120
PBP = PB + 8
JUNK = PB
NPB = NP * PB
SPT = 320
ZR = SPT

_mesh = plsc.VectorSubcoreMesh(core_axis_name="c", subcore_axis_name="s",
                               num_cores=NC)


def _make_segsum(with_deg):
  out_types = [jax.ShapeDtypeStruct((2 * NPB, D), jnp.float32)]
  if with_deg:
    out_types.append(jax.ShapeDtypeStruct((2 * NPB, D), jnp.float32))

  scratch = [
      pltpu.VMEM((TPC, CH), jnp.int32),
      pltpu.VMEM((TPC, CH), jnp.int32),
      pltpu.VMEM((CH,), jnp.int32),
      pltpu.VMEM((CH,), jnp.int32),
      pltpu.VMEM((CH, D), jnp.float32),
      pltpu.VMEM_SHARED((PBP, D), jnp.float32),
  ]
  if with_deg:
    scratch += [
        pltpu.VMEM((CH, D), jnp.float32),
    ]
  scratch.append(pltpu.SemaphoreType.DMA)

  def body(hu_hbm, hi_hbm, sui, dui, siu, diu, z128, z16, *rest):
    if with_deg:
      out_sum, out_deg = rest[0], rest[1]
      rest = rest[2:]
      src_v, dst_v, i1, i2, rows, accum, ones, sem = rest
    else:
      out_sum = rest[0]
      out_deg = None
      src_v, dst_v, i1, i2, rows, accum, sem = rest[1:]
      ones = None

    c = lax.axis_index("c")
    s = lax.axis_index("s")
    one16 = jnp.full((16,), 1.0, jnp.float32)

    if with_deg:
      @pl.loop(0, CH)
      def _(r):
        @pl.loop(0, D, step=16)
        def _(cc):
          ones[r, pl.ds(cc, 16)] = one16

    stripe = jnp.minimum(s * SPT, PB - SPT)


    def run(src3d, dst4d, h_hbm):
      pltpu.sync_copy(src3d.at[s], src_v)

      @pl.loop(0, NP)
      def _(p):
        pltpu.sync_copy(z128, accum.at[pl.ds(stripe, ZR)])
        pltpu.sync_copy(dst4d.at[p * NS + s], dst_v)
        plsc.subcore_barrier()

        @pl.loop(0, TPC)
        def _(j):
          for k in range(0, CH, 16):
            i1[pl.ds(k, 16)] = src_v[j, pl.ds(k, 16)]
            i2[pl.ds(k, 16)] = dst_v[j, pl.ds(k, 16)]
          pltpu.async_copy(h_hbm.at[i1], rows, sem).wait()
          pltpu.sync_copy(rows, accum.at[i2], add=True)

        plsc.subcore_barrier()
        pltpu.sync_copy(accum.at[pl.ds(stripe, SPT)],
                        out_sum.at[pl.ds(c * NPB + p * PB + stripe, SPT)])
        plsc.subcore_barrier()

      if with_deg:
        @pl.loop(0, NP)
        def _(p):
          pltpu.sync_copy(z128, accum.at[pl.ds(stripe, ZR)])
          pltpu.sync_copy(dst4d.at[p * NS + s], dst_v)
          plsc.subcore_barrier()

          @pl.loop(0, TPC)
          def _(j):
            for k in range(0, CH, 16):
              i2[pl.ds(k, 16)] = dst_v[j, pl.ds(k, 16)]
            pltpu.sync_copy(ones, accum.at[i2], add=True)

          plsc.subcore_barrier()
          pltpu.sync_copy(accum.at[pl.ds(stripe, SPT)],
                          out_deg.at[pl.ds(c * NPB + p * PB + stripe, SPT)])
          plsc.subcore_barrier()

    @pl.when(c == 0)
    def _():
      run(sui, dui, hu_hbm)

    @pl.when(c == 1)
    def _():
      run(siu, diu, hi_hbm)

  return pl.kernel(body, out_type=out_types, mesh=_mesh,
                   scratch_types=scratch)


_segsum_deg = _make_segsum(True)
_segsum = _make_segsum(False)


RB = 1000


def _proj_body(x_ref, w_ref, b_ref, o_ref):
  o_ref[...] = (jnp.dot(x_ref[...], w_ref[...],
                        preferred_element_type=jnp.float32) + b_ref[...])


def _proj(x, w, b):
  return pl.pallas_call(
      _proj_body,
      grid=(N // RB,),
      in_specs=[
          pl.BlockSpec((RB, D), lambda i: (i, 0)),
          pl.BlockSpec((D, D), lambda i: (0, 0)),
          pl.BlockSpec((D,), lambda i: (0,)),
      ],
      out_specs=pl.BlockSpec((RB, D), lambda i: (i, 0)),
      out_shape=jax.ShapeDtypeStruct((N, D), jnp.float32),
  )(x, w, b)


def _make_combine(apply_ln):
  def body(h_ref, summ_ref, deg_ref, ws_ref, wn_ref, b_ref, g_ref, bb_ref,
           o_ref):
    inv = 1.0 / jnp.maximum(deg_ref[...][:, 0:1], 1.0)
    h = (jnp.dot(h_ref[...], ws_ref[...], preferred_element_type=jnp.float32)
         + jnp.dot(summ_ref[...] * inv, wn_ref[...],
                   preferred_element_type=jnp.float32)
         + b_ref[...])
    if apply_ln:
      mu = jnp.mean(h, axis=-1, keepdims=True)
      var = jnp.mean((h - mu) ** 2, axis=-1, keepdims=True)
      h = g_ref[...] * (h - mu) / jnp.sqrt(var + 1e-5) + bb_ref[...]
      h = jnp.maximum(h, 0.0)
    o_ref[...] = h

  def call(h, summ, deg, ws, wn, b, g, bb):
    return pl.pallas_call(
        body,
        grid=(N // RB,),
        in_specs=[
            pl.BlockSpec((RB, D), lambda i: (i, 0)),
            pl.BlockSpec((RB, D), lambda i: (i, 0)),
            pl.BlockSpec((RB, D), lambda i: (i, 0)),
            pl.BlockSpec((D, D), lambda i: (0, 0)),
            pl.BlockSpec((D, D), lambda i: (0, 0)),
            pl.BlockSpec((D,), lambda i: (0,)),
            pl.BlockSpec((D,), lambda i: (0,)),
            pl.BlockSpec((D,), lambda i: (0,)),
        ],
        out_specs=pl.BlockSpec((RB, D), lambda i: (i, 0)),
        out_shape=jax.ShapeDtypeStruct((N, D), jnp.float32),
    )(h, summ, deg, ws, wn, b, g, bb)

  return call


_combine_ln = _make_combine(True)
_combine = _make_combine(False)


@jax.jit
def kernel(x_user, x_item, edge_rates, edge_rated_by,
           W_in_user, b_in_user, W_in_item, b_in_item,
           Wself0_ui, Wneigh0_ui, b0_ui, Wself0_iu, Wneigh0_iu, b0_iu,
           Wself1_ui, Wneigh1_ui, b1_ui, Wself1_iu, Wneigh1_iu, b1_iu,
           ln0_g, ln0_b):
  def pad(d, fill):
    return jnp.concatenate([d, jnp.full((EP - E,), fill, jnp.int32)])

  def remap(d):
    d = pad(d, N)
    slabs = [jnp.where((d >= k * PB) & (d < (k + 1) * PB), d - k * PB, JUNK)
             for k in range(NP)]
    return jnp.stack(slabs).reshape(NP * NS, TPC, CH)

  sui = pad(edge_rates[0], 0).reshape(NS, TPC, CH)
  dui = remap(edge_rates[1])
  siu = pad(edge_rated_by[0], 0).reshape(NS, TPC, CH)
  diu = remap(edge_rated_by[1])

  hu = _proj(x_user, W_in_user, b_in_user)
  hi = _proj(x_item, W_in_item, b_in_item)

  z128 = jnp.zeros((ZR, D), jnp.float32)
  z16 = jnp.zeros((ZR, DEGW), jnp.float32)

  summ0, deg = _segsum_deg(hu, hi, sui, dui, siu, diu, z128, z16)
  deg_i, deg_u = deg[:N], deg[NPB:NPB + N]

  hi_r = _combine_ln(hi, summ0[:N], deg_i, Wself0_ui, Wneigh0_ui, b0_ui,
                     ln0_g, ln0_b)
  hu_r = _combine_ln(hu, summ0[NPB:NPB + N], deg_u, Wself0_iu, Wneigh0_iu,
                     b0_iu, ln0_g, ln0_b)

  summ1, = _segsum(hu_r, hi_r, sui, dui, siu, diu, z128, z16)

  hi2 = _combine(hi_r, summ1[:N], deg_i, Wself1_ui, Wneigh1_ui, b1_ui,
                 ln0_g, ln0_b)
  hu2 = _combine(hu_r, summ1[NPB:NPB + N], deg_u, Wself1_iu, Wneigh1_iu,
                 b1_iu, ln0_g, ln0_b)
  return (hu2, hi2)


# speedup vs baseline: 3.2351x; 1.1862x over previous
import functools

import jax
import jax.numpy as jnp
from jax import lax
from jax.experimental import pallas as pl
from jax.experimental.pallas import tpu as pltpu
from jax.experimental.pallas import tpu_sc as plsc

N = 10000
D = 128
E = 320000

DEGW = 16
NC = 2
NS = 16
CH = 128
EP = 321536
TPC = EP // (NS * CH)
NP = 2
PB = 5000
PBP = PB + 8
JUNK = PB
NPB = NP * PB
SPT = 320
ZR = SPT

_mesh = plsc.VectorSubcoreMesh(core_axis_name="c", subcore_axis_name="s",
                               num_cores=NC)


def _make_segsum(with_deg):
  out_types = [jax.ShapeDtypeStruct((2 * NPB, D), jnp.float32)]
  if with_deg:
    out_types.append(jax.ShapeDtypeStruct((2 * NPB, D), jnp.float32))

  scratch = [
      pltpu.VMEM((TPC, CH), jnp.int32),
      pltpu.VMEM((TPC, CH), jnp.int32),
      pltpu.VMEM((CH,), jnp.int32),
      pltpu.VMEM((CH,), jnp.int32),
      pltpu.VMEM((CH,), jnp.int32),
      pltpu.VMEM((CH, D), jnp.float32),
      pltpu.VMEM((CH, D), jnp.float32),
      pltpu.VMEM_SHARED((PBP, D), jnp.float32),
  ]
  if with_deg:
    scratch += [
        pltpu.VMEM((CH, D), jnp.float32),
    ]
  scratch.append(pltpu.SemaphoreType.DMA)

  def body(hu_hbm, hi_hbm, sui, dui, siu, diu, z128, z16, *rest):
    if with_deg:
      out_sum, out_deg = rest[0], rest[1]
      rest = rest[2:]
      src_v, dst_v, i1a, i1b, i2, rowsa, rowsb, accum, ones, sem = rest
    else:
      out_sum = rest[0]
      out_deg = None
      src_v, dst_v, i1a, i1b, i2, rowsa, rowsb, accum, sem = rest[1:]
      ones = None

    c = lax.axis_index("c")
    s = lax.axis_index("s")
    one16 = jnp.full((16,), 1.0, jnp.float32)

    if with_deg:
      @pl.loop(0, CH)
      def _(r):
        @pl.loop(0, D, step=16)
        def _(cc):
          ones[r, pl.ds(cc, 16)] = one16

    stripe = jnp.minimum(s * SPT, PB - SPT)


    def run(src3d, dst4d, h_hbm):
      pltpu.sync_copy(src3d.at[s], src_v)

      @pl.loop(0, NP)
      def _(p):
        pltpu.sync_copy(z128, accum.at[pl.ds(stripe, ZR)])
        pltpu.sync_copy(dst4d.at[p * NS + s], dst_v)
        plsc.subcore_barrier()

        def fill(buf, vv, j):
          for k in range(0, CH, 16):
            buf[pl.ds(k, 16)] = vv[j, pl.ds(k, 16)]

        def scat(j, rbuf):
          fill(i2, dst_v, j)
          pltpu.sync_copy(rbuf, accum.at[i2], add=True)

        fill(i1a, src_v, 0)
        pltpu.async_copy(h_hbm.at[i1a], rowsa, sem)

        @pl.loop(0, TPC // 2)
        def _(p):
          j = 2 * p
          pltpu.make_async_copy(h_hbm.at[i1a], rowsa, sem).wait()
          fill(i1b, src_v, j + 1)
          pltpu.async_copy(h_hbm.at[i1b], rowsb, sem)
          scat(j, rowsa)
          pltpu.make_async_copy(h_hbm.at[i1b], rowsb, sem).wait()
          fill(i1a, src_v, j + 2)
          pltpu.async_copy(h_hbm.at[i1a], rowsa, sem)
          scat(j + 1, rowsb)

        pltpu.make_async_copy(h_hbm.at[i1a], rowsa, sem).wait()
        scat(TPC - 1, rowsa)

        plsc.subcore_barrier()
        pltpu.sync_copy(accum.at[pl.ds(stripe, SPT)],
                        out_sum.at[pl.ds(c * NPB + p * PB + stripe, SPT)])
        plsc.subcore_barrier()

      if with_deg:
        @pl.loop(0, NP)
        def _(p):
          pltpu.sync_copy(z128, accum.at[pl.ds(stripe, ZR)])
          pltpu.sync_copy(dst4d.at[p * NS + s], dst_v)
          plsc.subcore_barrier()

          @pl.loop(0, TPC)
          def _(j):
            for k in range(0, CH, 16):
              i2[pl.ds(k, 16)] = dst_v[j, pl.ds(k, 16)]
            pltpu.sync_copy(ones, accum.at[i2], add=True)

          plsc.subcore_barrier()
          pltpu.sync_copy(accum.at[pl.ds(stripe, SPT)],
                          out_deg.at[pl.ds(c * NPB + p * PB + stripe, SPT)])
          plsc.subcore_barrier()

    @pl.when(c == 0)
    def _():
      run(sui, dui, hu_hbm)

    @pl.when(c == 1)
    def _():
      run(siu, diu, hi_hbm)

  return pl.kernel(body, out_type=out_types, mesh=_mesh,
                   scratch_types=scratch)


_segsum_deg = _make_segsum(True)
_segsum = _make_segsum(False)


RB = 1000


def _proj_body(x_ref, w_ref, b_ref, o_ref):
  o_ref[...] = (jnp.dot(x_ref[...], w_ref[...],
                        preferred_element_type=jnp.float32) + b_ref[...])


def _proj(x, w, b):
  return pl.pallas_call(
      _proj_body,
      grid=(N // RB,),
      in_specs=[
          pl.BlockSpec((RB, D), lambda i: (i, 0)),
          pl.BlockSpec((D, D), lambda i: (0, 0)),
          pl.BlockSpec((D,), lambda i: (0,)),
      ],
      out_specs=pl.BlockSpec((RB, D), lambda i: (i, 0)),
      out_shape=jax.ShapeDtypeStruct((N, D), jnp.float32),
  )(x, w, b)


def _make_combine(apply_ln):
  def body(h_ref, summ_ref, deg_ref, ws_ref, wn_ref, b_ref, g_ref, bb_ref,
           o_ref):
    inv = 1.0 / jnp.maximum(deg_ref[...][:, 0:1], 1.0)
    h = (jnp.dot(h_ref[...], ws_ref[...], preferred_element_type=jnp.float32)
         + jnp.dot(summ_ref[...] * inv, wn_ref[...],
                   preferred_element_type=jnp.float32)
         + b_ref[...])
    if apply_ln:
      mu = jnp.mean(h, axis=-1, keepdims=True)
      var = jnp.mean((h - mu) ** 2, axis=-1, keepdims=True)
      h = g_ref[...] * (h - mu) / jnp.sqrt(var + 1e-5) + bb_ref[...]
      h = jnp.maximum(h, 0.0)
    o_ref[...] = h

  def call(h, summ, deg, ws, wn, b, g, bb):
    return pl.pallas_call(
        body,
        grid=(N // RB,),
        in_specs=[
            pl.BlockSpec((RB, D), lambda i: (i, 0)),
            pl.BlockSpec((RB, D), lambda i: (i, 0)),
            pl.BlockSpec((RB, D), lambda i: (i, 0)),
            pl.BlockSpec((D, D), lambda i: (0, 0)),
            pl.BlockSpec((D, D), lambda i: (0, 0)),
            pl.BlockSpec((D,), lambda i: (0,)),
            pl.BlockSpec((D,), lambda i: (0,)),
            pl.BlockSpec((D,), lambda i: (0,)),
        ],
        out_specs=pl.BlockSpec((RB, D), lambda i: (i, 0)),
        out_shape=jax.ShapeDtypeStruct((N, D), jnp.float32),
    )(h, summ, deg, ws, wn, b, g, bb)

  return call


_combine_ln = _make_combine(True)
_combine = _make_combine(False)


@jax.jit
def kernel(x_user, x_item, edge_rates, edge_rated_by,
           W_in_user, b_in_user, W_in_item, b_in_item,
           Wself0_ui, Wneigh0_ui, b0_ui, Wself0_iu, Wneigh0_iu, b0_iu,
           Wself1_ui, Wneigh1_ui, b1_ui, Wself1_iu, Wneigh1_iu, b1_iu,
           ln0_g, ln0_b):
  def pad(d, fill):
    return jnp.concatenate([d, jnp.full((EP - E,), fill, jnp.int32)])

  def remap(d):
    d = pad(d, N)
    slabs = [jnp.where((d >= k * PB) & (d < (k + 1) * PB), d - k * PB, JUNK)
             for k in range(NP)]
    return jnp.stack(slabs).reshape(NP * NS, TPC, CH)

  sui = pad(edge_rates[0], 0).reshape(NS, TPC, CH)
  dui = remap(edge_rates[1])
  siu = pad(edge_rated_by[0], 0).reshape(NS, TPC, CH)
  diu = remap(edge_rated_by[1])

  hu = _proj(x_user, W_in_user, b_in_user)
  hi = _proj(x_item, W_in_item, b_in_item)

  z128 = jnp.zeros((ZR, D), jnp.float32)
  z16 = jnp.zeros((ZR, DEGW), jnp.float32)

  summ0, deg = _segsum_deg(hu, hi, sui, dui, siu, diu, z128, z16)
  deg_i, deg_u = deg[:N], deg[NPB:NPB + N]

  hi_r = _combine_ln(hi, summ0[:N], deg_i, Wself0_ui, Wneigh0_ui, b0_ui,
                     ln0_g, ln0_b)
  hu_r = _combine_ln(hu, summ0[NPB:NPB + N], deg_u, Wself0_iu, Wneigh0_iu,
                     b0_iu, ln0_g, ln0_b)

  summ1, = _segsum(hu_r, hi_r, sui, dui, siu, diu, z128, z16)

  hi2 = _combine(hi_r, summ1[:N], deg_i, Wself1_ui, Wneigh1_ui, b1_ui,
                 ln0_g, ln0_b)
  hu2 = _combine(hu_r, summ1[NPB:NPB + N], deg_u, Wself1_iu, Wneigh1_iu,
                 b1_iu, ln0_g, ln0_b)
  return (hu2, hi2)


# speedup vs baseline: 3.2367x; 1.0005x over previous
import jax
import jax.numpy as jnp
from jax import lax
from jax.experimental import pallas as pl
from jax.experimental.pallas import tpu as pltpu
from jax.experimental.pallas import tpu_sc as plsc

N = 10000
D = 128
E = 320000

DEGW = 16
NC = 2
NS = 16
CH = 128
EP = 321536
TPC = EP // (NS * CH)
NP = 2
PB = 5000
PBP = PB + 8
JUNK = PB
NPB = NP * PB
SPT = 320
ZR = SPT

_mesh = plsc.VectorSubcoreMesh(core_axis_name="c", subcore_axis_name="s",
                               num_cores=NC)


def _make_segsum(with_deg):
  out_types = [jax.ShapeDtypeStruct((2 * NPB, D), jnp.float32)]
  if with_deg:
    out_types.append(jax.ShapeDtypeStruct((2 * NPB, D), jnp.float32))

  scratch = [
      pltpu.VMEM((TPC, CH), jnp.int32),
      pltpu.VMEM((TPC, CH), jnp.int32),
      pltpu.VMEM((CH,), jnp.int32),
      pltpu.VMEM((CH,), jnp.int32),
      pltpu.VMEM((CH,), jnp.int32),
      pltpu.VMEM((CH, D), jnp.float32),
      pltpu.VMEM((CH, D), jnp.float32),
      pltpu.VMEM_SHARED((PBP, D), jnp.float32),
  ]
  if with_deg:
    scratch += [
        pltpu.VMEM((CH, D), jnp.float32),
    ]
  scratch.append(pltpu.SemaphoreType.DMA)

  def body(hu_hbm, hi_hbm, sui, dui, siu, diu, z128, z16, *rest):
    if with_deg:
      out_sum, out_deg = rest[0], rest[1]
      rest = rest[2:]
      src_v, dst_v, i1a, i1b, i2, rowsa, rowsb, accum, ones, sem = rest
    else:
      out_sum = rest[0]
      out_deg = None
      src_v, dst_v, i1a, i1b, i2, rowsa, rowsb, accum, sem = rest[1:]
      ones = None

    c = lax.axis_index("c")
    s = lax.axis_index("s")
    one16 = jnp.full((16,), 1.0, jnp.float32)

    if with_deg:
      @pl.loop(0, CH)
      def _(r):
        @pl.loop(0, D, step=16)
        def _(cc):
          ones[r, pl.ds(cc, 16)] = one16

    stripe = jnp.minimum(s * SPT, PB - SPT)


    def run(src3d, dst4d, h_hbm):
      pltpu.sync_copy(src3d.at[s], src_v)

      @pl.loop(0, NP)
      def _(p):
        pltpu.sync_copy(z128, accum.at[pl.ds(stripe, ZR)])
        pltpu.sync_copy(dst4d.at[p * NS + s], dst_v)
        plsc.subcore_barrier()

        def fill(buf, vv, j):
          for k in range(0, CH, 16):
            buf[pl.ds(k, 16)] = vv[j, pl.ds(k, 16)]

        def scat(j, rbuf):
          fill(i2, dst_v, j)
          pltpu.sync_copy(rbuf, accum.at[i2], add=True)

        fill(i1a, src_v, 0)
        pltpu.async_copy(h_hbm.at[i1a], rowsa, sem)

        @pl.loop(0, TPC // 2)
        def _(p):
          j = 2 * p
          pltpu.make_async_copy(h_hbm.at[i1a], rowsa, sem).wait()
          fill(i1b, src_v, j + 1)
          pltpu.async_copy(h_hbm.at[i1b], rowsb, sem)
          scat(j, rowsa)
          pltpu.make_async_copy(h_hbm.at[i1b], rowsb, sem).wait()
          fill(i1a, src_v, j + 2)
          pltpu.async_copy(h_hbm.at[i1a], rowsa, sem)
          scat(j + 1, rowsb)

        pltpu.make_async_copy(h_hbm.at[i1a], rowsa, sem).wait()
        scat(TPC - 1, rowsa)

        plsc.subcore_barrier()
        pltpu.sync_copy(accum.at[pl.ds(stripe, SPT)],
                        out_sum.at[pl.ds(c * NPB + p * PB + stripe, SPT)])
        plsc.subcore_barrier()

      if with_deg:
        @pl.loop(0, NP)
        def _(p):
          pltpu.sync_copy(z128, accum.at[pl.ds(stripe, ZR)])
          pltpu.sync_copy(dst4d.at[p * NS + s], dst_v)
          plsc.subcore_barrier()

          @pl.loop(0, TPC)
          def _(j):
            for k in range(0, CH, 16):
              i2[pl.ds(k, 16)] = dst_v[j, pl.ds(k, 16)]
            pltpu.sync_copy(ones, accum.at[i2], add=True)

          plsc.subcore_barrier()
          pltpu.sync_copy(accum.at[pl.ds(stripe, SPT)],
                          out_deg.at[pl.ds(c * NPB + p * PB + stripe, SPT)])
          plsc.subcore_barrier()

    @pl.when(c == 0)
    def _():
      run(sui, dui, hu_hbm)

    @pl.when(c == 1)
    def _():
      run(siu, diu, hi_hbm)

  return pl.kernel(body, out_type=out_types, mesh=_mesh,
                   scratch_types=scratch)


_segsum_deg = _make_segsum(True)
_segsum = _make_segsum(False)


RB = 1000


def _proj_body(x_ref, w_ref, b_ref, o_ref):
  o_ref[...] = (jnp.dot(x_ref[...], w_ref[...],
                        preferred_element_type=jnp.float32) + b_ref[...])


def _proj(x, w, b):
  return pl.pallas_call(
      _proj_body,
      grid=(N // RB,),
      in_specs=[
          pl.BlockSpec((RB, D), lambda i: (i, 0)),
          pl.BlockSpec((D, D), lambda i: (0, 0)),
          pl.BlockSpec((D,), lambda i: (0,)),
      ],
      out_specs=pl.BlockSpec((RB, D), lambda i: (i, 0)),
      out_shape=jax.ShapeDtypeStruct((N, D), jnp.float32),
  )(x, w, b)


def _make_combine(apply_ln):
  def body(h_ref, summ_ref, deg_ref, ws_ref, wn_ref, b_ref, g_ref, bb_ref,
           o_ref):
    inv = 1.0 / jnp.maximum(deg_ref[...][:, 0:1], 1.0)
    h = (jnp.dot(h_ref[...], ws_ref[...], preferred_element_type=jnp.float32)
         + jnp.dot(summ_ref[...] * inv, wn_ref[...],
                   preferred_element_type=jnp.float32)
         + b_ref[...])
    if apply_ln:
      mu = jnp.mean(h, axis=-1, keepdims=True)
      var = jnp.mean((h - mu) ** 2, axis=-1, keepdims=True)
      h = g_ref[...] * (h - mu) / jnp.sqrt(var + 1e-5) + bb_ref[...]
      h = jnp.maximum(h, 0.0)
    o_ref[...] = h

  def call(h, summ, deg, ws, wn, b, g, bb):
    return pl.pallas_call(
        body,
        grid=(N // RB,),
        in_specs=[
            pl.BlockSpec((RB, D), lambda i: (i, 0)),
            pl.BlockSpec((RB, D), lambda i: (i, 0)),
            pl.BlockSpec((RB, D), lambda i: (i, 0)),
            pl.BlockSpec((D, D), lambda i: (0, 0)),
            pl.BlockSpec((D, D), lambda i: (0, 0)),
            pl.BlockSpec((D,), lambda i: (0,)),
            pl.BlockSpec((D,), lambda i: (0,)),
            pl.BlockSpec((D,), lambda i: (0,)),
        ],
        out_specs=pl.BlockSpec((RB, D), lambda i: (i, 0)),
        out_shape=jax.ShapeDtypeStruct((N, D), jnp.float32),
    )(h, summ, deg, ws, wn, b, g, bb)

  return call


_combine_ln = _make_combine(True)
_combine = _make_combine(False)


@jax.jit
def kernel(x_user, x_item, edge_rates, edge_rated_by,
           W_in_user, b_in_user, W_in_item, b_in_item,
           Wself0_ui, Wneigh0_ui, b0_ui, Wself0_iu, Wneigh0_iu, b0_iu,
           Wself1_ui, Wneigh1_ui, b1_ui, Wself1_iu, Wneigh1_iu, b1_iu,
           ln0_g, ln0_b):
  def pad(d, fill):
    return jnp.concatenate([d, jnp.full((EP - E,), fill, jnp.int32)])

  def remap(d):
    d = pad(d, N)
    slabs = [jnp.where((d >= k * PB) & (d < (k + 1) * PB), d - k * PB, JUNK)
             for k in range(NP)]
    return jnp.stack(slabs).reshape(NP * NS, TPC, CH)

  sui = pad(edge_rates[0], 0).reshape(NS, TPC, CH)
  dui = remap(edge_rates[1])
  siu = pad(edge_rated_by[0], 0).reshape(NS, TPC, CH)
  diu = remap(edge_rated_by[1])

  hu = _proj(x_user, W_in_user, b_in_user)
  hi = _proj(x_item, W_in_item, b_in_item)

  z128 = jnp.zeros((ZR, D), jnp.float32)
  z16 = jnp.zeros((ZR, DEGW), jnp.float32)

  summ0, deg = _segsum_deg(hu, hi, sui, dui, siu, diu, z128, z16)
  deg_i, deg_u = deg[:N], deg[NPB:NPB + N]

  hi_r = _combine_ln(hi, summ0[:N], deg_i, Wself0_ui, Wneigh0_ui, b0_ui,
                     ln0_g, ln0_b)
  hu_r = _combine_ln(hu, summ0[NPB:NPB + N], deg_u, Wself0_iu, Wneigh0_iu,
                     b0_iu, ln0_g, ln0_b)

  summ1, = _segsum(hu_r, hi_r, sui, dui, siu, diu, z128, z16)

  hi2 = _combine(hi_r, summ1[:N], deg_i, Wself1_ui, Wneigh1_ui, b1_ui,
                 ln0_g, ln0_b)
  hu2 = _combine(hu_r, summ1[NPB:NPB + N], deg_u, Wself1_iu, Wneigh1_iu,
                 b1_iu, ln0_g, ln0_b)
  return (hu2, hi2)
